# baseline (speedup 1.0000x reference)
import functools

import jax
import jax.numpy as jnp
from jax import lax
from jax.experimental import pallas as pl
from jax.experimental.pallas import tpu as pltpu
from jax.experimental.pallas import tpu_sc as plsc


def _sc_gather(table, idx):
    n = idx.shape[0]
    d = table.shape[1]
    info = plsc.get_sparse_core_info()
    nw = info.num_cores * info.num_subcores
    n_per_w = n // nw
    mesh = plsc.VectorSubcoreMesh(core_axis_name="c", subcore_axis_name="s")

    @functools.partial(
        pl.kernel,
        mesh=mesh,
        out_type=jax.ShapeDtypeStruct((n, d), jnp.float32),
        scratch_types=[
            pltpu.VMEM((n_per_w,), jnp.int32),
            pltpu.VMEM((n_per_w, d), jnp.float32),
            pltpu.SemaphoreType.DMA,
        ],
    )
    def gather_kernel(table_hbm, idx_hbm, out_hbm, idx_v, rows_v, sem):
        wid = lax.axis_index("s") * info.num_cores + lax.axis_index("c")
        base = wid * n_per_w
        pltpu.sync_copy(idx_hbm.at[pl.ds(base, n_per_w)], idx_v)
        pltpu.async_copy(table_hbm.at[idx_v], rows_v, sem).wait()
        pltpu.sync_copy(rows_v, out_hbm.at[pl.ds(base, n_per_w)])

    return gather_kernel(table, idx)


def _mm_body(a_ref, w_ref, b_ref, o_ref):
    o_ref[...] = (
        jnp.dot(a_ref[...], w_ref[...], preferred_element_type=jnp.float32)
        + b_ref[...]
    )


def _matmul_bias(a, w_t, bias, block_m=2048, block_n=1024):
    m, k = a.shape
    n = w_t.shape[1]
    return pl.pallas_call(
        _mm_body,
        grid=(m // block_m, n // block_n),
        in_specs=[
            pl.BlockSpec((block_m, k), lambda i, j: (i, 0)),
            pl.BlockSpec((k, block_n), lambda i, j: (0, j)),
            pl.BlockSpec((1, block_n), lambda i, j: (0, j)),
        ],
        out_specs=pl.BlockSpec((block_m, block_n), lambda i, j: (i, j)),
        out_shape=jax.ShapeDtypeStruct((m, n), jnp.float32),
    )(a, w_t, bias)


def _lstm_scan(xg, w_hh_t, batch_major_y=False):
    s, b, g4 = xg.shape
    h_dim = w_hh_t.shape[0]
    w_hh_t = w_hh_t.astype(jnp.bfloat16)

    def body(x_ref, w_ref, y_ref, h_ref, c_ref):
        t = pl.program_id(0)

        @pl.when(t == 0)
        def _():
            h_ref[...] = jnp.zeros_like(h_ref)
            c_ref[...] = jnp.zeros_like(c_ref)

        gates = x_ref[0] + jnp.dot(
            h_ref[...].astype(w_ref.dtype), w_ref[...],
            preferred_element_type=jnp.float32,
        )
        gi = jax.nn.sigmoid(gates[:, :h_dim])
        gf = jax.nn.sigmoid(gates[:, h_dim : 2 * h_dim])
        gg = jnp.tanh(gates[:, 2 * h_dim : 3 * h_dim])
        go = jax.nn.sigmoid(gates[:, 3 * h_dim :])
        c_new = gf * c_ref[...] + gi * gg
        h_new = go * jnp.tanh(c_new)
        if batch_major_y:
            y_ref[...] = h_new
        else:
            y_ref[0] = h_new
        h_ref[...] = h_new
        c_ref[...] = c_new

    if batch_major_y:
        y_spec = pl.BlockSpec((b, h_dim), lambda t: (0, t))
        y_shape = jax.ShapeDtypeStruct((b, s * h_dim), jnp.float32)
    else:
        y_spec = pl.BlockSpec((1, b, h_dim), lambda t: (t, 0, 0))
        y_shape = jax.ShapeDtypeStruct((s, b, h_dim), jnp.float32)

    return pl.pallas_call(
        body,
        grid=(s,),
        in_specs=[
            pl.BlockSpec((1, b, g4), lambda t: (t, 0, 0)),
            pl.BlockSpec((h_dim, g4), lambda t: (0, 0)),
        ],
        out_specs=[
            y_spec,
            pl.BlockSpec((b, h_dim), lambda t: (0, 0)),
            pl.BlockSpec((b, h_dim), lambda t: (0, 0)),
        ],
        out_shape=[
            y_shape,
            jax.ShapeDtypeStruct((b, h_dim), jnp.float32),
            jax.ShapeDtypeStruct((b, h_dim), jnp.float32),
        ],
    )(xg, w_hh_t)


def kernel(x, emb, W_ih0, W_hh0, b_ih0, b_hh0, W_ih1, W_hh1, b_ih1, b_hh1, W_out, b_out):
    b, s = x.shape
    d = emb.shape[1]
    h_dim = W_hh0.shape[1]

    idx = x.T.reshape(-1).astype(jnp.int32)
    e = _sc_gather(emb, idx)

    xg0 = _matmul_bias(e, W_ih0.T, (b_ih0 + b_hh0)[None, :])
    y0, h0, c0 = _lstm_scan(xg0.reshape(s, b, -1), W_hh0.T)

    xg1 = _matmul_bias(y0.reshape(s * b, h_dim), W_ih1.T, (b_ih1 + b_hh1)[None, :])
    y1, h1, c1 = _lstm_scan(xg1.reshape(s, b, -1), W_hh1.T)

    a = jnp.transpose(y1, (1, 0, 2)).reshape(b * s, h_dim)
    out = _matmul_bias(a, W_out.T, b_out[None, :]).reshape(b, s, -1)

    h_n = jnp.stack([h0, h1], axis=0)
    c_n = jnp.stack([c0, c1], axis=0)
    return (out, h_n, c_n)


# speedup vs baseline: 1.0589x; 1.0589x over previous
import functools

import jax
import jax.numpy as jnp
from jax import lax
from jax.experimental import pallas as pl
from jax.experimental.pallas import tpu as pltpu
from jax.experimental.pallas import tpu_sc as plsc


def _sc_gather(table, idx):
    n = idx.shape[0]
    d = table.shape[1]
    info = plsc.get_sparse_core_info()
    nw = info.num_cores * info.num_subcores
    n_per_w = n // nw
    mesh = plsc.VectorSubcoreMesh(core_axis_name="c", subcore_axis_name="s")

    @functools.partial(
        pl.kernel,
        mesh=mesh,
        out_type=jax.ShapeDtypeStruct((n, d), jnp.float32),
        scratch_types=[
            pltpu.VMEM((n_per_w,), jnp.int32),
            pltpu.VMEM((n_per_w, d), jnp.float32),
            pltpu.SemaphoreType.DMA,
        ],
    )
    def gather_kernel(table_hbm, idx_hbm, out_hbm, idx_v, rows_v, sem):
        wid = lax.axis_index("s") * info.num_cores + lax.axis_index("c")
        base = wid * n_per_w
        pltpu.sync_copy(idx_hbm.at[pl.ds(base, n_per_w)], idx_v)
        pltpu.async_copy(table_hbm.at[idx_v], rows_v, sem).wait()
        pltpu.sync_copy(rows_v, out_hbm.at[pl.ds(base, n_per_w)])

    return gather_kernel(table, idx)


def _mm_body(a_ref, w_ref, b_ref, o_ref):
    o_ref[...] = (
        jnp.dot(a_ref[...], w_ref[...], preferred_element_type=jnp.float32)
        + b_ref[...]
    )


def _matmul_bias(a, w_t, bias, block_m=2048, block_n=1024):
    m, k = a.shape
    n = w_t.shape[1]
    return pl.pallas_call(
        _mm_body,
        grid=(m // block_m, n // block_n),
        in_specs=[
            pl.BlockSpec((block_m, k), lambda i, j: (i, 0)),
            pl.BlockSpec((k, block_n), lambda i, j: (0, j)),
            pl.BlockSpec((1, block_n), lambda i, j: (0, j)),
        ],
        out_specs=pl.BlockSpec((block_m, block_n), lambda i, j: (i, j)),
        out_shape=jax.ShapeDtypeStruct((m, n), jnp.float32),
    )(a, w_t, bias)


def _lstm_scan(xg, w_hh_t, unroll=4):
    s, b, g4 = xg.shape
    h_dim = w_hh_t.shape[0]
    w_hh_t = w_hh_t.astype(jnp.bfloat16)

    def body(x_ref, w_ref, y_ref, h_ref, c_ref, hb_ref):
        t = pl.program_id(0)

        @pl.when(t == 0)
        def _():
            h_ref[...] = jnp.zeros_like(h_ref)
            c_ref[...] = jnp.zeros_like(c_ref)
            hb_ref[...] = jnp.zeros_like(hb_ref)

        h_bf = hb_ref[...]
        c = c_ref[...]
        for u in range(unroll):
            gates = x_ref[u] + jnp.dot(
                h_bf, w_ref[...], preferred_element_type=jnp.float32
            )
            gi = jax.nn.sigmoid(gates[:, :h_dim])
            gf = jax.nn.sigmoid(gates[:, h_dim : 2 * h_dim])
            gg = jnp.tanh(gates[:, 2 * h_dim : 3 * h_dim])
            go = jax.nn.sigmoid(gates[:, 3 * h_dim :])
            c = gf * c + gi * gg
            h_new = go * jnp.tanh(c)
            y_ref[u] = h_new
            h_bf = h_new.astype(jnp.bfloat16)
        h_ref[...] = h_new
        c_ref[...] = c
        hb_ref[...] = h_bf

    return pl.pallas_call(
        body,
        grid=(s // unroll,),
        in_specs=[
            pl.BlockSpec((unroll, b, g4), lambda t: (t, 0, 0)),
            pl.BlockSpec((h_dim, g4), lambda t: (0, 0)),
        ],
        out_specs=[
            pl.BlockSpec((unroll, b, h_dim), lambda t: (t, 0, 0)),
            pl.BlockSpec((b, h_dim), lambda t: (0, 0)),
            pl.BlockSpec((b, h_dim), lambda t: (0, 0)),
        ],
        out_shape=[
            jax.ShapeDtypeStruct((s, b, h_dim), jnp.float32),
            jax.ShapeDtypeStruct((b, h_dim), jnp.float32),
            jax.ShapeDtypeStruct((b, h_dim), jnp.float32),
        ],
        scratch_shapes=[pltpu.VMEM((b, h_dim), jnp.bfloat16)],
    )(xg, w_hh_t)


def kernel(x, emb, W_ih0, W_hh0, b_ih0, b_hh0, W_ih1, W_hh1, b_ih1, b_hh1, W_out, b_out):
    b, s = x.shape
    d = emb.shape[1]
    h_dim = W_hh0.shape[1]

    idx = x.T.reshape(-1).astype(jnp.int32)
    e = _sc_gather(emb, idx)

    xg0 = _matmul_bias(e, W_ih0.T, (b_ih0 + b_hh0)[None, :])
    y0, h0, c0 = _lstm_scan(xg0.reshape(s, b, -1), W_hh0.T)

    xg1 = _matmul_bias(y0.reshape(s * b, h_dim), W_ih1.T, (b_ih1 + b_hh1)[None, :])
    y1, h1, c1 = _lstm_scan(xg1.reshape(s, b, -1), W_hh1.T)

    a = jnp.transpose(y1, (1, 0, 2)).reshape(b * s, h_dim)
    out = _matmul_bias(a, W_out.T, b_out[None, :]).reshape(b, s, -1)

    h_n = jnp.stack([h0, h1], axis=0)
    c_n = jnp.stack([c0, c1], axis=0)
    return (out, h_n, c_n)


# speedup vs baseline: 1.0777x; 1.0177x over previous
import functools

import jax
import jax.numpy as jnp
from jax import lax
from jax.experimental import pallas as pl
from jax.experimental.pallas import tpu as pltpu
from jax.experimental.pallas import tpu_sc as plsc


def _sc_gather(table, idx):
    n = idx.shape[0]
    d = table.shape[1]
    info = plsc.get_sparse_core_info()
    nw = info.num_cores * info.num_subcores
    n_per_w = n // nw
    mesh = plsc.VectorSubcoreMesh(core_axis_name="c", subcore_axis_name="s")

    @functools.partial(
        pl.kernel,
        mesh=mesh,
        out_type=jax.ShapeDtypeStruct((n, d), jnp.float32),
        scratch_types=[
            pltpu.VMEM((n_per_w,), jnp.int32),
            pltpu.VMEM((n_per_w, d), jnp.float32),
            pltpu.SemaphoreType.DMA,
        ],
    )
    def gather_kernel(table_hbm, idx_hbm, out_hbm, idx_v, rows_v, sem):
        wid = lax.axis_index("s") * info.num_cores + lax.axis_index("c")
        base = wid * n_per_w
        pltpu.sync_copy(idx_hbm.at[pl.ds(base, n_per_w)], idx_v)
        pltpu.async_copy(table_hbm.at[idx_v], rows_v, sem).wait()
        pltpu.sync_copy(rows_v, out_hbm.at[pl.ds(base, n_per_w)])

    return gather_kernel(table, idx)


def _mm_body(a_ref, w_ref, b_ref, o_ref):
    acc = (
        jnp.dot(
            a_ref[...].astype(jnp.bfloat16),
            w_ref[...],
            preferred_element_type=jnp.float32,
        )
        + b_ref[...]
    )
    o_ref[...] = acc.astype(o_ref.dtype)


def _matmul_bias(a, w_t, bias, block_m=2048, block_n=1024, out_dtype=jnp.float32):
    m, k = a.shape
    n = w_t.shape[1]
    return pl.pallas_call(
        _mm_body,
        grid=(m // block_m, n // block_n),
        in_specs=[
            pl.BlockSpec((block_m, k), lambda i, j: (i, 0)),
            pl.BlockSpec((k, block_n), lambda i, j: (0, j)),
            pl.BlockSpec((1, block_n), lambda i, j: (0, j)),
        ],
        out_specs=pl.BlockSpec((block_m, block_n), lambda i, j: (i, j)),
        out_shape=jax.ShapeDtypeStruct((m, n), out_dtype),
    )(a, w_t.astype(jnp.bfloat16), bias)


def _lstm_scan(xg, w_hh_t, unroll=8):
    s, b, g4 = xg.shape
    h_dim = w_hh_t.shape[0]
    w_hh_t = w_hh_t.astype(jnp.bfloat16)

    def body(x_ref, w_ref, y_ref, h_ref, c_ref, hb_ref):
        t = pl.program_id(0)

        @pl.when(t == 0)
        def _():
            h_ref[...] = jnp.zeros_like(h_ref)
            c_ref[...] = jnp.zeros_like(c_ref)
            hb_ref[...] = jnp.zeros_like(hb_ref)

        h_bf = hb_ref[...]
        c = c_ref[...]
        for u in range(unroll):
            gates = x_ref[u] + jnp.dot(
                h_bf, w_ref[...], preferred_element_type=jnp.float32
            )
            gi = jax.nn.sigmoid(gates[:, :h_dim])
            gf = jax.nn.sigmoid(gates[:, h_dim : 2 * h_dim])
            gg = jnp.tanh(gates[:, 2 * h_dim : 3 * h_dim])
            go = jax.nn.sigmoid(gates[:, 3 * h_dim :])
            c = gf * c + gi * gg
            h_new = go * jnp.tanh(c)
            y_ref[u] = h_new
            h_bf = h_new.astype(jnp.bfloat16)
        h_ref[...] = h_new
        c_ref[...] = c
        hb_ref[...] = h_bf

    return pl.pallas_call(
        body,
        grid=(s // unroll,),
        in_specs=[
            pl.BlockSpec((unroll, b, g4), lambda t: (t, 0, 0)),
            pl.BlockSpec((h_dim, g4), lambda t: (0, 0)),
        ],
        out_specs=[
            pl.BlockSpec((unroll, b, h_dim), lambda t: (t, 0, 0)),
            pl.BlockSpec((b, h_dim), lambda t: (0, 0)),
            pl.BlockSpec((b, h_dim), lambda t: (0, 0)),
        ],
        out_shape=[
            jax.ShapeDtypeStruct((s, b, h_dim), jnp.float32),
            jax.ShapeDtypeStruct((b, h_dim), jnp.float32),
            jax.ShapeDtypeStruct((b, h_dim), jnp.float32),
        ],
        scratch_shapes=[pltpu.VMEM((b, h_dim), jnp.bfloat16)],
    )(xg, w_hh_t)


def kernel(x, emb, W_ih0, W_hh0, b_ih0, b_hh0, W_ih1, W_hh1, b_ih1, b_hh1, W_out, b_out):
    b, s = x.shape
    d = emb.shape[1]
    h_dim = W_hh0.shape[1]

    idx = x.T.reshape(-1).astype(jnp.int32)
    e = _sc_gather(emb, idx)

    xg0 = _matmul_bias(e, W_ih0.T, (b_ih0 + b_hh0)[None, :], out_dtype=jnp.bfloat16)
    y0, h0, c0 = _lstm_scan(xg0.reshape(s, b, -1), W_hh0.T)

    xg1 = _matmul_bias(y0.reshape(s * b, h_dim), W_ih1.T, (b_ih1 + b_hh1)[None, :], out_dtype=jnp.bfloat16)
    y1, h1, c1 = _lstm_scan(xg1.reshape(s, b, -1), W_hh1.T)

    a = jnp.transpose(y1, (1, 0, 2)).reshape(b * s, h_dim)
    out = _matmul_bias(a, W_out.T, b_out[None, :]).reshape(b, s, -1)

    h_n = jnp.stack([h0, h1], axis=0)
    c_n = jnp.stack([c0, c1], axis=0)
    return (out, h_n, c_n)
